# Initial kernel scaffold; baseline (speedup 1.0000x reference)
import jax
import jax.numpy as jnp
from jax.experimental import pallas as pl


def kernel(x, edge_index, W1, b1, W2, b2):
    raise NotImplementedError("write your pallas kernel here")


# speedup vs baseline: 12.6753x; 12.6753x over previous
import functools

import jax
import jax.numpy as jnp
from jax import lax
from jax.experimental import pallas as pl
from jax.experimental.pallas import tpu as pltpu
from jax.experimental.pallas import tpu_sc as plsc

N = 10000
E = 320000
D_IN = 128
D_H = 128
D_OUT = 64

NC = 2
NS = 16
NW = NC * NS

K = 128
CH = -(-E // (NW * K))
EP = NW * K * CH
NPAD = 10240
RPT = NPAD // NS

RB = 1000
NB = N // RB


def _sc_mesh():
    return plsc.VectorSubcoreMesh(
        core_axis_name="c", subcore_axis_name="s", num_cores=NC, num_subcores=NS)


@functools.cache
def _make_sc_deg():
    @functools.partial(
        pl.kernel,
        out_type=jax.ShapeDtypeStruct((NC, NPAD), jnp.float32),
        mesh=_sc_mesh(),
        scratch_types=[
            pltpu.VMEM((K,), jnp.int32),
            pltpu.VMEM((K,), jnp.float32),
            pltpu.VMEM((RPT,), jnp.float32),
            pltpu.VMEM_SHARED((NPAD,), jnp.float32),
        ],
    )
    def _sc_deg(dst_hbm, out_hbm, dst_v, ones_v, zbuf, acc):
        cid = lax.axis_index("c")
        sid = lax.axis_index("s")
        w = cid * NS + sid

        zeros16 = jnp.zeros((16,), jnp.float32)
        ones16 = jnp.ones((16,), jnp.float32)

        @pl.loop(0, K // 16)
        def _fill(i):
            ones_v[pl.ds(i * 16, 16)] = ones16

        @pl.loop(0, RPT // 16)
        def _zb(i):
            zbuf[pl.ds(i * 16, 16)] = zeros16

        pltpu.sync_copy(zbuf, acc.at[pl.ds(sid * RPT, RPT)])
        plsc.subcore_barrier()

        base = w * CH * K

        @pl.loop(0, CH)
        def _edges(t):
            pltpu.sync_copy(dst_hbm.at[pl.ds(base + t * K, K)], dst_v)
            pltpu.sync_copy(ones_v, acc.at[dst_v], add=True)

        plsc.subcore_barrier()
        pltpu.sync_copy(acc.at[pl.ds(sid * RPT, RPT)],
                        out_hbm.at[cid, pl.ds(sid * RPT, RPT)])

    return _sc_deg


@functools.cache
def _make_sc_agg(d):
    @functools.partial(
        pl.kernel,
        out_type=jax.ShapeDtypeStruct((NC, NPAD, d), jnp.float32),
        mesh=_sc_mesh(),
        scratch_types=[
            pltpu.VMEM((K,), jnp.int32),
            pltpu.VMEM((K,), jnp.int32),
            pltpu.VMEM((K, d), jnp.float32),
            pltpu.VMEM_SHARED((NPAD, d), jnp.float32),
            pltpu.SemaphoreType.DMA,
        ],
        compiler_params=pltpu.CompilerParams(use_tc_tiling_on_sc=False),
    )
    def agg(h_hbm, src_hbm, dst_hbm, out_hbm, src_v, dst_v, rows_v, acc, sem):
        cid = lax.axis_index("c")
        sid = lax.axis_index("s")
        w = cid * NS + sid

        zeros16 = jnp.zeros((16,), jnp.float32)

        @pl.loop(0, K)
        def _zr(i):
            for j in range(d // 16):
                rows_v[i, pl.ds(j * 16, 16)] = zeros16

        for t in range(RPT // K):
            pltpu.sync_copy(rows_v, acc.at[pl.ds(sid * RPT + t * K, K)])
        plsc.subcore_barrier()

        base = w * CH * K

        @pl.loop(0, CH)
        def _edges(t):
            off = base + t * K
            pltpu.sync_copy(src_hbm.at[pl.ds(off, K)], src_v)
            pltpu.sync_copy(dst_hbm.at[pl.ds(off, K)], dst_v)
            pltpu.async_copy(h_hbm.at[src_v], rows_v, sem).wait()
            pltpu.sync_copy(rows_v, acc.at[dst_v], add=True)

        plsc.subcore_barrier()
        pltpu.sync_copy(acc.at[pl.ds(sid * RPT, RPT)],
                        out_hbm.at[cid, pl.ds(sid * RPT, RPT)])

    return agg


def _tc1_body(dp_ref, x_ref, w1_ref, h1_ref, dis_ref):
    deg = dp_ref[0, 0] + dp_ref[0, 1] + 1.0
    dis = lax.rsqrt(deg)
    h = jnp.dot(x_ref[...], w1_ref[...], preferred_element_type=jnp.float32)
    h1_ref[...] = dis[:, None] * h
    dis_ref[0, 0] = dis


def _tc1(degp3, x, W1):
    return pl.pallas_call(
        _tc1_body,
        grid=(NB,),
        in_specs=[
            pl.BlockSpec((1, NC, RB), lambda i: (i, 0, 0)),
            pl.BlockSpec((RB, D_IN), lambda i: (i, 0)),
            pl.BlockSpec((D_IN, D_H), lambda i: (0, 0)),
        ],
        out_specs=[
            pl.BlockSpec((RB, D_H), lambda i: (i, 0)),
            pl.BlockSpec((1, 1, RB), lambda i: (i, 0, 0)),
        ],
        out_shape=[
            jax.ShapeDtypeStruct((N, D_H), jnp.float32),
            jax.ShapeDtypeStruct((NB, 1, RB), jnp.float32),
        ],
    )(degp3, x, W1)


def _tc2_body(p_ref, h1_ref, dis_ref, w2_ref, b1_ref, out_ref):
    dis = dis_ref[0, 0]
    s = p_ref[0] + p_ref[1] + h1_ref[...]
    h = jnp.maximum(dis[:, None] * s + b1_ref[...][None, :], 0.0)
    out_ref[...] = dis[:, None] * jnp.dot(
        h, w2_ref[...], preferred_element_type=jnp.float32)


def _tc2(parts1, h1p, dis2, W2, b1):
    return pl.pallas_call(
        _tc2_body,
        grid=(NB,),
        in_specs=[
            pl.BlockSpec((2, RB, D_H), lambda i: (0, i, 0)),
            pl.BlockSpec((RB, D_H), lambda i: (i, 0)),
            pl.BlockSpec((1, 1, RB), lambda i: (i, 0, 0)),
            pl.BlockSpec((D_H, D_OUT), lambda i: (0, 0)),
            pl.BlockSpec((D_H,), lambda i: (0,)),
        ],
        out_specs=pl.BlockSpec((RB, D_OUT), lambda i: (i, 0)),
        out_shape=jax.ShapeDtypeStruct((N, D_OUT), jnp.float32),
    )(parts1, h1p, dis2, W2, b1)


def _tc3_body(p_ref, h2_ref, dis_ref, b2_ref, out_ref):
    dis = dis_ref[0, 0]
    s = p_ref[0] + p_ref[1] + h2_ref[...]
    out_ref[...] = dis[:, None] * s + b2_ref[...][None, :]


def _tc3(parts2, h2p, dis2, b2):
    return pl.pallas_call(
        _tc3_body,
        grid=(NB,),
        in_specs=[
            pl.BlockSpec((2, RB, D_OUT), lambda i: (0, i, 0)),
            pl.BlockSpec((RB, D_OUT), lambda i: (i, 0)),
            pl.BlockSpec((1, 1, RB), lambda i: (i, 0, 0)),
            pl.BlockSpec((D_OUT,), lambda i: (0,)),
        ],
        out_specs=pl.BlockSpec((RB, D_OUT), lambda i: (i, 0)),
        out_shape=jax.ShapeDtypeStruct((N, D_OUT), jnp.float32),
    )(parts2, h2p, dis2, b2)


def kernel(x, edge_index, W1, b1, W2, b2):
    pad = EP - E
    src_p = jnp.concatenate([edge_index[0], jnp.zeros((pad,), jnp.int32)])
    dst_p = jnp.concatenate([edge_index[1], jnp.full((pad,), N, jnp.int32)])

    degp = _make_sc_deg()(dst_p)
    degp3 = degp[:, :N].reshape(NC, NB, RB).transpose(1, 0, 2)

    h1p, dis2 = _tc1(degp3, x, W1)
    parts1 = _make_sc_agg(D_H)(h1p, src_p, dst_p)
    h2p = _tc2(parts1, h1p, dis2, W2, b1)
    parts2 = _make_sc_agg(D_OUT)(h2p, src_p, dst_p)
    return _tc3(parts2, h2p, dis2, b2)
